# Initial kernel scaffold; baseline (speedup 1.0000x reference)
import jax
import jax.numpy as jnp
from jax.experimental import pallas as pl


def kernel(entity_emb, user_emb, edge_index, edge_type, relation_weight, kg_W_r):
    raise NotImplementedError("write your pallas kernel here")


# speedup vs baseline: 6.6887x; 6.6887x over previous
import functools
import jax
import jax.numpy as jnp
from jax import lax
from jax.experimental import pallas as pl
from jax.experimental.pallas import tpu as pltpu
from jax.experimental.pallas import tpu_sc as plsc

N = 10000
E = 320000
C = 128
R = 32

NC = 2
NS = 16
NW = NC * NS
NP = 10240
EP = 327680
EW_CH = EP // NW
SUB = 128
NSUB = EW_CH // SUB
NODES_W = NP // NW
BLK = 2000

_mesh = plsc.VectorSubcoreMesh(
    core_axis_name="c", subcore_axis_name="s", num_cores=NC, num_subcores=NS)


def _wid():
    return lax.axis_index("s") * NC + lax.axis_index("c")


_LANES = lambda: lax.iota(jnp.int32, 16)


def _fill(ref, n, value):
    lanes = _LANES()
    v = jnp.full((16,), value, ref.dtype)

    def body(i, _):
        plsc.store_scatter(ref, [lanes + i * 16], v)
        return 0

    lax.fori_loop(0, n // 16, body, 0)


def _seg_max_update(acc_ref, idx, val):
    def body(_):
        cur = plsc.load_gather(acc_ref, [idx])
        need = val > cur
        plsc.store_scatter(acc_ref, [idx], val, mask=need)
        cur2 = plsc.load_gather(acc_ref, [idx])
        return jnp.any(val > cur2)

    lax.while_loop(lambda again: again, lambda _: body(None), body(None))


def _seg_add_update(acc_ref, lane_ref, idx, val):
    lanes = _LANES()

    def cond(act):
        return jnp.any(act)

    def body(act):
        plsc.store_scatter(lane_ref, [idx], lanes, mask=act)
        back = plsc.load_gather(lane_ref, [idx])
        win = jnp.logical_and(act, back == lanes)
        cur = plsc.load_gather(acc_ref, [idx])
        plsc.store_scatter(acc_ref, [idx], cur + val, mask=win)
        return jnp.logical_and(act, jnp.logical_not(win))

    lax.while_loop(cond, body, jnp.ones((16,), jnp.bool_))


def _tc_body(e_ref, rw_ref, w_ref, s_ref, rn_ref):
    w = w_ref[...]
    rww = jnp.dot(rw_ref[...], w, preferred_element_type=jnp.float32)
    ew = jnp.dot(e_ref[...], w, preferred_element_type=jnp.float32)
    s_ref[...] = lax.dot_general(
        ew, rww, (((1,), (1,)), ((), ())),
        preferred_element_type=jnp.float32) * 0.0625

    @pl.when(pl.program_id(0) == 0)
    def _():
        rn_ref[...] = jnp.sum(rw_ref[...] * rw_ref[...], axis=1)[None, :]


def _tc_scores(entity_emb, relation_weight, kg_W_r):
    return pl.pallas_call(
        _tc_body,
        grid=(N // BLK,),
        in_specs=[
            pl.BlockSpec((BLK, C), lambda i: (i, 0)),
            pl.BlockSpec((R, C), lambda i: (0, 0)),
            pl.BlockSpec((C, C), lambda i: (0, 0)),
        ],
        out_specs=[
            pl.BlockSpec((BLK, R), lambda i: (i, 0)),
            pl.BlockSpec((1, R), lambda i: (0, 0)),
        ],
        out_shape=[
            jax.ShapeDtypeStruct((N, R), jnp.float32),
            jax.ShapeDtypeStruct((1, R), jnp.float32),
        ],
    )(entity_emb, relation_weight, kg_W_r)


@functools.partial(
    pl.kernel,
    out_type=[
        jax.ShapeDtypeStruct((EP,), jnp.float32),
        jax.ShapeDtypeStruct((NW * NP,), jnp.float32),
    ],
    mesh=_mesh,
    compiler_params=pltpu.CompilerParams(needs_layout_passes=False),
    scratch_types=[
        pltpu.VMEM((EW_CH,), jnp.int32),
        pltpu.VMEM((EW_CH,), jnp.int32),
        pltpu.VMEM((EW_CH,), jnp.float32),
        pltpu.VMEM((NP,), jnp.float32),
        pltpu.VMEM((NP,), jnp.int32),
        pltpu.SemaphoreType.DMA,
    ],
)
def _sc_pass1(flatidx_hbm, head_hbm, sflat_hbm, ex1_hbm, d1p_hbm,
              idx_v, head_v, val_v, acc_v, lane_v, sem):
    wid = _wid()
    base = wid * EW_CH
    pltpu.sync_copy(flatidx_hbm.at[pl.ds(base, EW_CH)], idx_v)
    pltpu.sync_copy(head_hbm.at[pl.ds(base, EW_CH)], head_v)

    def fire(c, _):
        o = pl.multiple_of(c * SUB, SUB)
        pltpu.make_async_copy(
            sflat_hbm.at[idx_v.at[pl.ds(o, SUB)]],
            val_v.at[pl.ds(o, SUB)], sem).start()
        return 0

    lax.fori_loop(0, NSUB, fire, 0)

    def drain(c, _):
        o = pl.multiple_of(c * SUB, SUB)
        pltpu.make_async_copy(
            sflat_hbm.at[idx_v.at[pl.ds(o, SUB)]],
            val_v.at[pl.ds(o, SUB)], sem).wait()
        return 0

    lax.fori_loop(0, NSUB, drain, 0)

    _fill(acc_v, NP, 0.0)
    lanes = _LANES()

    def upd(g, _):
        eid = lanes + g * 16
        h = plsc.load_gather(head_v, [eid])
        ex = jnp.exp(plsc.load_gather(val_v, [eid]))
        plsc.store_scatter(val_v, [eid], ex)
        _seg_add_update(acc_v, lane_v, h, ex)
        return 0

    lax.fori_loop(0, EW_CH // 16, upd, 0)

    pltpu.sync_copy(val_v, ex1_hbm.at[pl.ds(base, EW_CH)])
    pltpu.sync_copy(acc_v, d1p_hbm.at[pl.ds(pl.multiple_of(wid * NP, 8), NP)])


def _combine_body(op, init, p_hbm, out_hbm, buf_v, acc_v):
    wid = _wid()
    nb = wid * NODES_W
    _fill(acc_v, NODES_W, init)
    lanes = _LANES()

    def per_worker(a, _):
        pltpu.sync_copy(
            p_hbm.at[pl.ds(pl.multiple_of(a * NP + nb, 8), NODES_W)], buf_v)

        def red(i, _):
            ii = lanes + i * 16
            cur = plsc.load_gather(acc_v, [ii])
            add = plsc.load_gather(buf_v, [ii])
            plsc.store_scatter(acc_v, [ii], op(cur, add))
            return 0

        lax.fori_loop(0, NODES_W // 16, red, 0)
        return 0

    lax.fori_loop(0, NW, per_worker, 0)
    pltpu.sync_copy(acc_v, out_hbm.at[pl.ds(nb, NODES_W)])


_combine_scratch = [
    pltpu.VMEM((NODES_W,), jnp.float32),
    pltpu.VMEM((NODES_W,), jnp.float32),
]


@functools.partial(
    pl.kernel, out_type=jax.ShapeDtypeStruct((NP,), jnp.float32),
    mesh=_mesh, scratch_types=_combine_scratch,
    compiler_params=pltpu.CompilerParams(needs_layout_passes=False))
def _sc_combine_sum(p_hbm, out_hbm, buf_v, acc_v):
    _combine_body(lambda a, b: a + b, 0.0, p_hbm, out_hbm, buf_v, acc_v)


@functools.partial(
    pl.kernel, out_type=jax.ShapeDtypeStruct((NP,), jnp.float32),
    mesh=_mesh, scratch_types=_combine_scratch,
    compiler_params=pltpu.CompilerParams(needs_layout_passes=False))
def _sc_combine_max(p_hbm, out_hbm, buf_v, acc_v):
    _combine_body(jnp.maximum, -jnp.inf, p_hbm, out_hbm, buf_v, acc_v)


@functools.partial(
    pl.kernel,
    out_type=[
        jax.ShapeDtypeStruct((EP,), jnp.float32),
        jax.ShapeDtypeStruct((NW * NP,), jnp.float32),
    ],
    mesh=_mesh,
    compiler_params=pltpu.CompilerParams(needs_layout_passes=False),
    scratch_types=[
        pltpu.VMEM((EW_CH,), jnp.int32),
        pltpu.VMEM((EW_CH,), jnp.int32),
        pltpu.VMEM((EW_CH,), jnp.int32),
        pltpu.VMEM((EW_CH,), jnp.float32),
        pltpu.VMEM((NP,), jnp.float32),
        pltpu.VMEM((NP,), jnp.float32),
        pltpu.VMEM((R,), jnp.float32),
        pltpu.VMEM((2, SUB, C), jnp.float32),
        pltpu.VMEM((2, SUB, C), jnp.float32),
        pltpu.SemaphoreType.DMA((2,)),
        pltpu.SemaphoreType.DMA((2,)),
    ],
)
def _sc_pass2(head_hbm, tail_hbm, ridx_hbm, ex1_hbm, d1_hbm, emb_hbm, rn_hbm,
              st_hbm, m2p_hbm,
              head_v, tail_v, ridx_v, val_v, d1_v, acc_v, rn_v,
              hrow, trow, sem_h, sem_t):
    wid = _wid()
    base = wid * EW_CH
    pltpu.sync_copy(head_hbm.at[pl.ds(base, EW_CH)], head_v)
    pltpu.sync_copy(tail_hbm.at[pl.ds(base, EW_CH)], tail_v)
    pltpu.sync_copy(ridx_hbm.at[pl.ds(base, EW_CH)], ridx_v)
    pltpu.sync_copy(ex1_hbm.at[pl.ds(base, EW_CH)], val_v)
    pltpu.sync_copy(d1_hbm, d1_v)
    pltpu.sync_copy(rn_hbm, rn_v)
    _fill(acc_v, NP, -jnp.inf)
    lanes = _LANES()

    def prefetch(b, c):
        o = pl.multiple_of(c * SUB, SUB)
        pltpu.make_async_copy(
            emb_hbm.at[head_v.at[pl.ds(o, SUB)]], hrow.at[b],
            sem_h.at[b]).start()
        pltpu.make_async_copy(
            emb_hbm.at[tail_v.at[pl.ds(o, SUB)]], trow.at[b],
            sem_t.at[b]).start()

    def compute(b, c):
        o = pl.multiple_of(c * SUB, SUB)
        pltpu.make_async_copy(
            emb_hbm.at[head_v.at[pl.ds(o, SUB)]], hrow.at[b],
            sem_h.at[b]).wait()
        pltpu.make_async_copy(
            emb_hbm.at[tail_v.at[pl.ds(o, SUB)]], trow.at[b],
            sem_t.at[b]).wait()
        for g in range(SUB // 16):
            le = lanes + g * 16
            ge = le + o
            h16 = plsc.load_gather(head_v, [ge])
            e116 = plsc.load_gather(val_v, [ge])
            d1g = plsc.load_gather(d1_v, [h16])
            rel = e116 / (d1g + 1e-16)
            rn16 = plsc.load_gather(rn_v, [plsc.load_gather(ridx_v, [ge])])

            def dch(c8, dot):
                for u in range(8):
                    cvec = jnp.full((16,), 0, jnp.int32) + (c8 * 8 + u)
                    hv = plsc.load_gather(hrow.at[b], [le, cvec])
                    tv = plsc.load_gather(trow.at[b], [le, cvec])
                    dot = dot + hv * tv
                return dot

            dot = lax.fori_loop(0, C // 8, dch, jnp.zeros((16,), jnp.float32))
            st16 = dot + rel * rel * rn16
            plsc.store_scatter(val_v, [ge], st16)
            _seg_max_update(acc_v, h16, st16)

    prefetch(0, 0)

    def outer(i, _):
        c0 = i * 2
        for b in range(2):
            c = c0 + b

            @pl.when(c + 1 < NSUB)
            def _():
                prefetch(1 - b, c + 1)

            compute(b, c)
        return 0

    lax.fori_loop(0, NSUB // 2, outer, 0)

    pltpu.sync_copy(val_v, st_hbm.at[pl.ds(base, EW_CH)])
    pltpu.sync_copy(acc_v, m2p_hbm.at[pl.ds(pl.multiple_of(wid * NP, 8), NP)])


@functools.partial(
    pl.kernel,
    out_type=[
        jax.ShapeDtypeStruct((EP,), jnp.float32),
        jax.ShapeDtypeStruct((NW * NP,), jnp.float32),
    ],
    mesh=_mesh,
    compiler_params=pltpu.CompilerParams(needs_layout_passes=False),
    scratch_types=[
        pltpu.VMEM((EW_CH,), jnp.int32),
        pltpu.VMEM((EW_CH,), jnp.float32),
        pltpu.VMEM((NP,), jnp.float32),
        pltpu.VMEM((NP,), jnp.float32),
        pltpu.VMEM((NP,), jnp.int32),
    ],
)
def _sc_pass3(head_hbm, st_hbm, m2_hbm, ex2_hbm, d2p_hbm,
              head_v, val_v, m2_v, acc_v, lane_v):
    wid = _wid()
    base = wid * EW_CH
    pltpu.sync_copy(head_hbm.at[pl.ds(base, EW_CH)], head_v)
    pltpu.sync_copy(st_hbm.at[pl.ds(base, EW_CH)], val_v)
    pltpu.sync_copy(m2_hbm, m2_v)
    _fill(acc_v, NP, 0.0)
    lanes = _LANES()

    def upd(g, _):
        eid = lanes + g * 16
        h = plsc.load_gather(head_v, [eid])
        st = plsc.load_gather(val_v, [eid])
        ex = jnp.exp(st - plsc.load_gather(m2_v, [h]))
        plsc.store_scatter(val_v, [eid], ex)
        _seg_add_update(acc_v, lane_v, h, ex)
        return 0

    lax.fori_loop(0, EW_CH // 16, upd, 0)

    pltpu.sync_copy(val_v, ex2_hbm.at[pl.ds(base, EW_CH)])
    pltpu.sync_copy(acc_v, d2p_hbm.at[pl.ds(pl.multiple_of(wid * NP, 8), NP)])


@functools.partial(
    pl.kernel,
    out_type=jax.ShapeDtypeStruct((EP,), jnp.float32),
    mesh=_mesh,
    compiler_params=pltpu.CompilerParams(needs_layout_passes=False),
    scratch_types=[
        pltpu.VMEM((EW_CH,), jnp.int32),
        pltpu.VMEM((EW_CH,), jnp.float32),
        pltpu.VMEM((NP,), jnp.float32),
    ],
)
def _sc_pass4(head_hbm, ex2_hbm, d2_hbm, out_hbm, head_v, val_v, d2_v):
    wid = _wid()
    base = wid * EW_CH
    pltpu.sync_copy(head_hbm.at[pl.ds(base, EW_CH)], head_v)
    pltpu.sync_copy(ex2_hbm.at[pl.ds(base, EW_CH)], val_v)
    pltpu.sync_copy(d2_hbm, d2_v)
    lanes = _LANES()

    def upd(g, _):
        eid = lanes + g * 16
        h = plsc.load_gather(head_v, [eid])
        ex = plsc.load_gather(val_v, [eid])
        d2g = plsc.load_gather(d2_v, [h])
        plsc.store_scatter(val_v, [eid], ex / (d2g + 1e-16))
        return 0

    lax.fori_loop(0, EW_CH // 16, upd, 0)
    pltpu.sync_copy(val_v, out_hbm.at[pl.ds(base, EW_CH)])


def kernel(entity_emb, user_emb, edge_index, edge_type, relation_weight,
           kg_W_r):
    del user_emb
    head = edge_index[0].astype(jnp.int32)
    tail = edge_index[1].astype(jnp.int32)
    ridx = ((edge_type.astype(jnp.int32) - 1) % R).astype(jnp.int32)

    pad = EP - E
    headp = jnp.concatenate([head, jnp.full((pad,), NP - 1, jnp.int32)])
    tailp = jnp.concatenate([tail, jnp.zeros((pad,), jnp.int32)])
    ridxp = jnp.concatenate([ridx, jnp.zeros((pad,), jnp.int32)])
    flatidx = jnp.concatenate(
        [head * R + ridx, jnp.zeros((pad,), jnp.int32)])

    s_mat, rn_mat = _tc_scores(entity_emb, relation_weight, kg_W_r)
    sflat = s_mat.reshape(-1)
    rnorm = rn_mat.reshape(-1)

    ex1, d1p = _sc_pass1(flatidx, headp, sflat)
    d1 = _sc_combine_sum(d1p)
    st, m2p = _sc_pass2(headp, tailp, ridxp, ex1, d1, entity_emb, rnorm)
    m2 = _sc_combine_max(m2p)
    ex2, d2p = _sc_pass3(headp, st, m2)
    d2 = _sc_combine_sum(d2p)
    out = _sc_pass4(headp, ex2, d2)
    return out[:E]
